# Initial kernel scaffold; baseline (speedup 1.0000x reference)
import jax
import jax.numpy as jnp
from jax.experimental import pallas as pl


def kernel(node_feat, src, dst, node_ids, W1, b1, g1, be1, W2, b2, g2, be2, W3, b3, Wn, bn, Wo, bo):
    raise NotImplementedError("write your pallas kernel here")


# speedup vs baseline: 12.0666x; 12.0666x over previous
import functools

import jax
import jax.numpy as jnp
from jax import lax
from jax.experimental import pallas as pl
from jax.experimental.pallas import tpu as pltpu
from jax.experimental.pallas import tpu_sc as plsc

N = 10000
E = 320000
D = 128
NPAD = 10240
NCORE = 2
NSUB = 16
NW = NCORE * NSUB
EPW = E // NW
EBLK = 80
NBLK = EPW // EBLK
RPT = NPAD // NSUB
CCH = RPT // EBLK
GPW = NPAD // NW

_mesh = plsc.VectorSubcoreMesh(core_axis_name="c", subcore_axis_name="s")
_sc_params = pltpu.CompilerParams(needs_layout_passes=False)


@functools.partial(
    pl.kernel,
    out_type=jax.ShapeDtypeStruct((NW * N,), jnp.float32),
    mesh=_mesh,
    scratch_types=[
        pltpu.VMEM((EPW,), jnp.int32),
        pltpu.VMEM((N,), jnp.float32),
    ],
    compiler_params=_sc_params,
)
def _deg_kernel(dst_hbm, out_hbm, idx_v, acc_v):
    w = lax.axis_index("s") * NCORE + lax.axis_index("c")
    pltpu.sync_copy(dst_hbm.at[pl.ds(w * EPW, EPW)], idx_v)
    zeros = jnp.zeros((16,), jnp.float32)
    ones = jnp.ones((16,), jnp.float32)

    def _zero(i, carry):
        acc_v[pl.ds(i * 16, 16)] = zeros
        return carry

    lax.fori_loop(0, N // 16, _zero, 0)

    def _count(i, carry):
        iv = idx_v[pl.ds(i * 16, 16)]
        plsc.addupdate_scatter(acc_v, [iv], ones)
        return carry

    lax.fori_loop(0, EPW // 16, _count, 0)
    pltpu.sync_copy(acc_v, out_hbm.at[pl.ds(w * N, N)])


@functools.partial(
    pl.kernel,
    out_type=jax.ShapeDtypeStruct((NCORE * NPAD, D), jnp.float32),
    mesh=_mesh,
    scratch_types=[
        pltpu.VMEM((EBLK,), jnp.int32),
        pltpu.VMEM((EBLK,), jnp.int32),
        pltpu.VMEM((EBLK, D), jnp.float32),
        pltpu.VMEM_SHARED((NPAD, D), jnp.float32),
        pltpu.SemaphoreType.DMA,
    ],
    compiler_params=_sc_params,
)
def _prop_kernel(hs_hbm, src_hbm, dst_hbm, out_hbm, src_v, dst_v, rows_v,
                 acc_sh, sem):
    c = lax.axis_index("c")
    s = lax.axis_index("s")
    w = s * NCORE + c
    zeros = jnp.zeros((16,), jnp.float32)

    def _zero(k, carry):
        i = k // (D // 16)
        j = k % (D // 16)
        rows_v[i, pl.ds(j * 16, 16)] = zeros
        return carry

    lax.fori_loop(0, EBLK * (D // 16), _zero, 0)
    for b in range(CCH):
        pltpu.sync_copy(rows_v, acc_sh.at[pl.ds(s * RPT + b * EBLK, EBLK)])
    plsc.subcore_barrier()

    def _edge_block(i, carry):
        off = w * EPW + i * EBLK
        pltpu.sync_copy(src_hbm.at[pl.ds(off, EBLK)], src_v)
        pltpu.sync_copy(dst_hbm.at[pl.ds(off, EBLK)], dst_v)
        pltpu.async_copy(hs_hbm.at[src_v], rows_v, sem).wait()
        pltpu.sync_copy(rows_v, acc_sh.at[dst_v], add=True)
        return carry

    lax.fori_loop(0, NBLK, _edge_block, 0)
    plsc.subcore_barrier()
    for b in range(CCH):
        r0 = s * RPT + b * EBLK
        pltpu.sync_copy(acc_sh.at[pl.ds(r0, EBLK)], rows_v)
        pltpu.sync_copy(rows_v, out_hbm.at[pl.ds(c * NPAD + r0, EBLK)])


@functools.partial(
    pl.kernel,
    out_type=jax.ShapeDtypeStruct((NPAD,), jnp.float32),
    mesh=_mesh,
    scratch_types=[
        pltpu.VMEM((NPAD,), jnp.float32),
        pltpu.VMEM((GPW,), jnp.int32),
        pltpu.VMEM((GPW,), jnp.float32),
    ],
    compiler_params=_sc_params,
)
def _gather_kernel(tab_hbm, ids_hbm, out_hbm, tab_v, idx_v, out_v):
    w = lax.axis_index("s") * NCORE + lax.axis_index("c")
    pltpu.sync_copy(tab_hbm, tab_v)
    pltpu.sync_copy(ids_hbm.at[pl.ds(w * GPW, GPW)], idx_v)

    def _g(i, carry):
        iv = idx_v[pl.ds(i * 16, 16)]
        out_v[pl.ds(i * 16, 16)] = plsc.load_gather(tab_v, [iv])
        return carry

    lax.fori_loop(0, GPW // 16, _g, 0)
    pltpu.sync_copy(out_v, out_hbm.at[pl.ds(w * GPW, GPW)])


def _first_body(degp_ref, x_ref, w1_ref, dinv_ref, hs_ref):
    degp = degp_ref[...]
    ones = jnp.ones((NW, 1), jnp.float32)
    deg = lax.dot_general(degp, ones, (((0,), (0,)), ((), ())),
                          preferred_element_type=jnp.float32)
    dinv = lax.rsqrt(deg + 1.0)
    dinv_ref[...] = dinv
    h = jnp.dot(x_ref[...], w1_ref[...], preferred_element_type=jnp.float32)
    hs_ref[...] = h * dinv


def _mid_body(p_ref, hs_ref, dinv_ref, b_ref, g_ref, be_ref, wn_ref, out_ref):
    dinv = dinv_ref[...]
    pm = p_ref[0:N, :] + p_ref[NPAD:NPAD + N, :]
    y = dinv * (pm + hs_ref[...]) + b_ref[...][None, :]
    mu = jnp.mean(y, axis=0, keepdims=True)
    var = jnp.mean((y - mu) * (y - mu), axis=0, keepdims=True)
    xn = (y - mu) * lax.rsqrt(var + 1e-5) * g_ref[...][None, :] \
        + be_ref[...][None, :]
    xn = jnp.maximum(xn, 0.0)
    h = jnp.dot(xn, wn_ref[...], preferred_element_type=jnp.float32)
    out_ref[...] = h * dinv


def _last_body(p_ref, hs_ref, dinv_ref, b3_ref, wn_ref, bn_ref, wo_ref,
               bo_ref, out_ref):
    dinv = dinv_ref[...]
    pm = p_ref[0:N, :] + p_ref[NPAD:NPAD + N, :]
    z = dinv * (pm + hs_ref[...]) + b3_ref[...][None, :]
    h = jnp.dot(z, wn_ref[...], preferred_element_type=jnp.float32)
    h = jnp.maximum(h + bn_ref[...][None, :], 0.0)
    pred = jnp.sum(h * wo_ref[...], axis=1) + bo_ref[...]
    out_ref[...] = pred


_first_tc = pl.pallas_call(
    _first_body,
    out_shape=(
        jax.ShapeDtypeStruct((N, 1), jnp.float32),
        jax.ShapeDtypeStruct((N, D), jnp.float32),
    ),
)

_mid_tc = pl.pallas_call(
    _mid_body,
    out_shape=jax.ShapeDtypeStruct((N, D), jnp.float32),
)

_last_tc = pl.pallas_call(
    _last_body,
    out_shape=jax.ShapeDtypeStruct((N,), jnp.float32),
)


def kernel(node_feat, src, dst, node_ids, W1, b1, g1, be1, W2, b2, g2, be2,
           W3, b3, Wn, bn, Wo, bo):
    degp = _deg_kernel(dst).reshape(NW, N)
    dinv, hs1 = _first_tc(degp, node_feat, W1)

    p1 = _prop_kernel(hs1, src, dst)
    hs2 = _mid_tc(p1, hs1, dinv, b1, g1, be1, W2)
    p2 = _prop_kernel(hs2, src, dst)
    hs3 = _mid_tc(p2, hs2, dinv, b2, g2, be2, W3)
    p3 = _prop_kernel(hs3, src, dst)
    predf = _last_tc(p3, hs3, dinv, b3, Wn, bn, Wo.reshape(1, D), bo)

    predf_pad = jnp.pad(predf, (0, NPAD - N))
    ids_pad = jnp.pad(node_ids, (0, NPAD - N))
    out = _gather_kernel(predf_pad, ids_pad)
    return out[:N].reshape(N, 1)
